# baseline (speedup 1.0000x reference)
import functools

import jax
import jax.numpy as jnp
from jax import lax
from jax.experimental import pallas as pl
from jax.experimental.pallas import tpu as pltpu
from jax.experimental.pallas import tpu_sc as plsc

VOCAB = 1000
D_MODEL = 128
BATCH = 1024
SEQ = 50
NUM_CORES = 2
NUM_SUBCORES = 16
NW = NUM_CORES * NUM_SUBCORES
STRIPES = (15, 35)
CHUNK = 80


@functools.cache
def _make_gather_rows(s0, sseq):
    tokens = BATCH * sseq
    tpw = tokens // NW
    nchunk = tpw // CHUNK
    assert nchunk * CHUNK == tpw and nchunk % 2 == 0
    tok0 = BATCH * s0
    mesh = plsc.VectorSubcoreMesh(core_axis_name="c", subcore_axis_name="s")

    @functools.partial(
        pl.kernel,
        mesh=mesh,
        out_type=jax.ShapeDtypeStruct((tokens, D_MODEL), jnp.float32),
        scratch_types=[
            pltpu.VMEM((tpw,), jnp.int32),
            pltpu.VMEM((CHUNK, D_MODEL), jnp.float32),
            pltpu.VMEM((CHUNK, D_MODEL), jnp.float32),
            pltpu.SemaphoreType.DMA,
            pltpu.SemaphoreType.DMA,
            pltpu.SemaphoreType.DMA,
            pltpu.SemaphoreType.DMA,
        ],
    )
    def _gather_rows(tab_hbm, idx_hbm, out_hbm, idx_v, buf_a, buf_b,
                     sg_a, sg_b, ss_a, ss_b):
        wid = lax.axis_index("s") * NUM_CORES + lax.axis_index("c")
        base = wid * tpw
        pltpu.sync_copy(idx_hbm.at[pl.ds(tok0 + base, tpw)], idx_v)

        bufs = (buf_a, buf_b)
        gsems = (sg_a, sg_b)
        ssems = (ss_a, ss_b)

        def gather_src(c):
            return tab_hbm.at[idx_v.at[pl.ds(c * CHUNK, CHUNK)]]

        def out_dst(c):
            return out_hbm.at[pl.ds(base + c * CHUNK, CHUNK)]

        pltpu.async_copy(gather_src(0), bufs[0], gsems[0])

        def body(c0, carry):
            for p in range(2):
                c = c0 + p
                buf, gs, ss = bufs[p], gsems[p], ssems[p]
                other = 1 - p
                pltpu.make_async_copy(gather_src(c), buf, gs).wait()
                nxt = c + 1

                @pl.when(nxt < nchunk)
                def _():
                    @pl.when(c >= 1)
                    def _():
                        pltpu.make_async_copy(
                            bufs[other], out_dst(c - 1), ssems[other]
                        ).wait()

                    pltpu.async_copy(gather_src(nxt), bufs[other], gsems[other])

                pltpu.async_copy(buf, out_dst(c), ss)
            return carry

        lax.fori_loop(0, nchunk // 2, lambda i, c: body(i * 2, c), 0)

        pltpu.make_async_copy(bufs[0], out_dst(nchunk - 2), ssems[0]).wait()
        pltpu.make_async_copy(bufs[1], out_dst(nchunk - 1), ssems[1]).wait()

    return _gather_rows


def _proj_kernel_first(wt_ref, e_ref, b_ref, o_ref):
    o_ref[0] = (
        lax.dot_general(
            wt_ref[...], e_ref[0],
            (((1,), (1,)), ((), ())),
            preferred_element_type=jnp.float32,
        )
        + b_ref[...]
    )


def _proj_kernel_next(wt_ref, e_ref, b_ref, prev_ref, o_ref):
    del prev_ref
    _proj_kernel_first(wt_ref, e_ref, b_ref, o_ref)


def _project_stripe(wt, emb_t, b_col, s0, sseq, prev):
    in_specs = [
        pl.BlockSpec((VOCAB, D_MODEL), lambda s: (0, 0)),
        pl.BlockSpec((1, BATCH, D_MODEL), lambda s: (s, 0, 0)),
        pl.BlockSpec((VOCAB, 1), lambda s: (0, 0)),
    ]
    args = [wt, emb_t, b_col]
    if prev is None:
        body = _proj_kernel_first
        aliases = {}
    else:
        body = _proj_kernel_next
        in_specs.append(pl.BlockSpec(memory_space=pl.ANY))
        args.append(prev)
        aliases = {3: 0}
    return pl.pallas_call(
        body,
        grid=(sseq,),
        in_specs=in_specs,
        out_specs=pl.BlockSpec((1, VOCAB, BATCH), lambda s: (s + s0, 0, 0)),
        out_shape=jax.ShapeDtypeStruct((SEQ, VOCAB, BATCH), jnp.float32),
        input_output_aliases=aliases,
    )(*args)


def kernel(x, emb_table, W, b):
    xt = x.astype(jnp.int32).T.reshape(SEQ * BATCH)
    wt, b_col = W.T, b[:, None]
    embs = []
    s0 = 0
    for sseq in STRIPES:
        emb_g = _make_gather_rows(s0, sseq)(emb_table, xt)
        embs.append((s0, sseq, emb_g.reshape(sseq, BATCH, D_MODEL)))
        s0 += sseq
    t = None
    for s0, sseq, emb_t in embs:
        t = _project_stripe(wt, emb_t, b_col, s0, sseq, t)
    return jnp.transpose(t, (2, 0, 1))


# speedup vs baseline: 1.0242x; 1.0242x over previous
import functools

import jax
import jax.numpy as jnp
from jax import lax
from jax.experimental import pallas as pl
from jax.experimental.pallas import tpu as pltpu
from jax.experimental.pallas import tpu_sc as plsc

VOCAB = 1000
D_MODEL = 128
BATCH = 1024
SEQ = 50
NUM_CORES = 2
NUM_SUBCORES = 16
NW = NUM_CORES * NUM_SUBCORES
STRIPES = (20, 30)
CHUNK = 80


@functools.cache
def _make_gather_rows(s0, sseq):
    tokens = BATCH * sseq
    tpw = tokens // NW
    nchunk = tpw // CHUNK
    assert nchunk * CHUNK == tpw and nchunk % 2 == 0
    tok0 = BATCH * s0
    mesh = plsc.VectorSubcoreMesh(core_axis_name="c", subcore_axis_name="s")

    @functools.partial(
        pl.kernel,
        mesh=mesh,
        out_type=jax.ShapeDtypeStruct((tokens, D_MODEL), jnp.float32),
        scratch_types=[
            pltpu.VMEM((tpw,), jnp.int32),
            pltpu.VMEM((CHUNK, D_MODEL), jnp.float32),
            pltpu.VMEM((CHUNK, D_MODEL), jnp.float32),
            pltpu.SemaphoreType.DMA,
            pltpu.SemaphoreType.DMA,
            pltpu.SemaphoreType.DMA,
            pltpu.SemaphoreType.DMA,
        ],
    )
    def _gather_rows(tab_hbm, idx_hbm, out_hbm, idx_v, buf_a, buf_b,
                     sg_a, sg_b, ss_a, ss_b):
        wid = lax.axis_index("s") * NUM_CORES + lax.axis_index("c")
        base = wid * tpw
        pltpu.sync_copy(idx_hbm.at[pl.ds(tok0 + base, tpw)], idx_v)

        bufs = (buf_a, buf_b)
        gsems = (sg_a, sg_b)
        ssems = (ss_a, ss_b)

        def gather_src(c):
            return tab_hbm.at[idx_v.at[pl.ds(c * CHUNK, CHUNK)]]

        def out_dst(c):
            return out_hbm.at[pl.ds(base + c * CHUNK, CHUNK)]

        pltpu.async_copy(gather_src(0), bufs[0], gsems[0])

        def body(c0, carry):
            for p in range(2):
                c = c0 + p
                buf, gs, ss = bufs[p], gsems[p], ssems[p]
                other = 1 - p
                pltpu.make_async_copy(gather_src(c), buf, gs).wait()
                nxt = c + 1

                @pl.when(nxt < nchunk)
                def _():
                    @pl.when(c >= 1)
                    def _():
                        pltpu.make_async_copy(
                            bufs[other], out_dst(c - 1), ssems[other]
                        ).wait()

                    pltpu.async_copy(gather_src(nxt), bufs[other], gsems[other])

                pltpu.async_copy(buf, out_dst(c), ss)
            return carry

        lax.fori_loop(0, nchunk // 2, lambda i, c: body(i * 2, c), 0)

        pltpu.make_async_copy(bufs[0], out_dst(nchunk - 2), ssems[0]).wait()
        pltpu.make_async_copy(bufs[1], out_dst(nchunk - 1), ssems[1]).wait()

    return _gather_rows


def _proj_kernel_first(wt_ref, e_ref, b_ref, o_ref):
    o_ref[0] = (
        lax.dot_general(
            wt_ref[...], e_ref[0],
            (((1,), (1,)), ((), ())),
            preferred_element_type=jnp.float32,
        )
        + b_ref[...]
    )


def _proj_kernel_next(wt_ref, e_ref, b_ref, prev_ref, o_ref):
    del prev_ref
    _proj_kernel_first(wt_ref, e_ref, b_ref, o_ref)


def _project_stripe(wt, emb_t, b_col, s0, sseq, prev):
    in_specs = [
        pl.BlockSpec((VOCAB, D_MODEL), lambda s: (0, 0)),
        pl.BlockSpec((1, BATCH, D_MODEL), lambda s: (s, 0, 0)),
        pl.BlockSpec((VOCAB, 1), lambda s: (0, 0)),
    ]
    args = [wt, emb_t, b_col]
    if prev is None:
        body = _proj_kernel_first
        aliases = {}
    else:
        body = _proj_kernel_next
        in_specs.append(pl.BlockSpec(memory_space=pl.ANY))
        args.append(prev)
        aliases = {3: 0}
    return pl.pallas_call(
        body,
        grid=(sseq,),
        in_specs=in_specs,
        out_specs=pl.BlockSpec((1, VOCAB, BATCH), lambda s: (s + s0, 0, 0)),
        out_shape=jax.ShapeDtypeStruct((SEQ, VOCAB, BATCH), jnp.float32),
        input_output_aliases=aliases,
    )(*args)


def kernel(x, emb_table, W, b):
    xt = x.astype(jnp.int32).T.reshape(SEQ * BATCH)
    wt, b_col = W.T, b[:, None]
    embs = []
    s0 = 0
    for sseq in STRIPES:
        emb_g = _make_gather_rows(s0, sseq)(emb_table, xt)
        embs.append((s0, sseq, emb_g.reshape(sseq, BATCH, D_MODEL)))
        s0 += sseq
    t = None
    for s0, sseq, emb_t in embs:
        t = _project_stripe(wt, emb_t, b_col, s0, sseq, t)
    return jnp.transpose(t, (2, 0, 1))
